# baseline (speedup 1.0000x reference)
import functools

import jax
import jax.numpy as jnp
from jax import lax
from jax.experimental import pallas as pl
from jax.experimental.pallas import tpu as pltpu
from jax.experimental.pallas import tpu_sc as plsc

VOCAB = 100000
EMBED_DIM = 64
CONTEXT = 20
BATCH = 4096
HIDDEN = 128

_NC = 2
_NS = 16
_NW = _NC * _NS

_CH = 128
_DPAD = 128

_VPAD = 100352
_TVA = 1024
_VJ = 16
_TVB = _VPAD // _VJ
_BT = 256

_LOG2E = 1.4426950408889634
_LN2 = 0.6931471805599453


def _sc_gather(idx_flat, emb_pad):
    n_idx = idx_flat.shape[0]
    d = emb_pad.shape[1]
    b_per_w = n_idx // _NW
    n_ch = b_per_w // _CH

    mesh = plsc.VectorSubcoreMesh(core_axis_name="c", subcore_axis_name="s")

    @functools.partial(
        pl.kernel,
        mesh=mesh,
        out_type=jax.ShapeDtypeStruct((n_idx, d), jnp.float32),
        scratch_types=[
            pltpu.VMEM((b_per_w,), jnp.int32),
            pltpu.VMEM((_CH, d), jnp.float32),
            pltpu.VMEM((_CH, d), jnp.float32),
            pltpu.SemaphoreType.DMA,
            pltpu.SemaphoreType.DMA,
        ],
    )
    def gather_kernel(table_hbm, idx_hbm, out_hbm, idx_v, buf0, buf1,
                      sem0, sem1):
        wid = lax.axis_index("s") * _NC + lax.axis_index("c")
        base = wid * b_per_w
        pltpu.sync_copy(idx_hbm.at[pl.ds(base, b_per_w)], idx_v)
        bufs = (buf0, buf1)
        sems = (sem0, sem1)
        copies = [None] * n_ch
        copies[0] = pltpu.async_copy(
            table_hbm.at[idx_v.at[pl.ds(0, _CH)]], bufs[0], sems[0])
        for t in range(n_ch):
            copies[t].wait()
            if t + 1 < n_ch:
                nb = (t + 1) % 2
                copies[t + 1] = pltpu.async_copy(
                    table_hbm.at[idx_v.at[pl.ds((t + 1) * _CH, _CH)]],
                    bufs[nb], sems[nb])
            pltpu.sync_copy(bufs[t % 2],
                            out_hbm.at[pl.ds(base + t * _CH, _CH)])

    return gather_kernel(emb_pad, idx_flat)


def _hidden_kernel(e_ref, w1_ref, b1_ref, h_ref, shift_ref):
    e = e_ref[...].astype(jnp.bfloat16)
    acc = lax.dot_general(e, w1_ref[...],
                          (((1,), (1,)), ((), ())),
                          preferred_element_type=jnp.float32)
    h = jnp.maximum(acc + b1_ref[...], 0.0)
    h_ref[...] = h.astype(jnp.bfloat16)
    bound = (jnp.sum(h, axis=1, keepdims=True) + 1.0) * (HIDDEN ** -0.5)
    shift_ref[...] = bound * _LOG2E


def _compute_hidden(embeds, w1b, b1):
    bt = 1024
    return pl.pallas_call(
        _hidden_kernel,
        grid=(BATCH // bt,),
        in_specs=[
            pl.BlockSpec((bt, CONTEXT * _DPAD), lambda i: (i, 0)),
            pl.BlockSpec((HIDDEN, CONTEXT * _DPAD), lambda i: (0, 0)),
            pl.BlockSpec((1, HIDDEN), lambda i: (0, 0)),
        ],
        out_specs=[
            pl.BlockSpec((bt, HIDDEN), lambda i: (i, 0)),
            pl.BlockSpec((bt, 1), lambda i: (i, 0)),
        ],
        out_shape=[
            jax.ShapeDtypeStruct((BATCH, HIDDEN), jnp.bfloat16),
            jax.ShapeDtypeStruct((BATCH, 1), jnp.float32),
        ],
    )(embeds, w1b, b1.reshape(1, HIDDEN))


def _lse_kernel(h_ref, w2_ref, b2_ref, shift_ref, lse_ref, s_s):
    j = pl.program_id(0)

    @pl.when(j == 0)
    def _init():
        s_s[...] = jnp.zeros((BATCH, 1), jnp.float32)

    d = lax.dot_general(h_ref[...], w2_ref[...],
                        (((1,), (1,)), ((), ())),
                        preferred_element_type=jnp.float32)
    e = jnp.exp2(d + b2_ref[...] - shift_ref[...])
    s_s[...] += jnp.sum(e, axis=1, keepdims=True)

    @pl.when(j == pl.num_programs(0) - 1)
    def _fin():
        lse_ref[...] = shift_ref[...] + jnp.log2(s_s[...])


def _compute_lse(h, w2s, b2s, shift):
    return pl.pallas_call(
        _lse_kernel,
        grid=(_VPAD // _TVA,),
        in_specs=[
            pl.BlockSpec((BATCH, HIDDEN), lambda j: (0, 0)),
            pl.BlockSpec((_TVA, HIDDEN), lambda j: (j, 0)),
            pl.BlockSpec((1, _TVA), lambda j: (0, j)),
            pl.BlockSpec((BATCH, 1), lambda j: (0, 0)),
        ],
        out_specs=pl.BlockSpec((BATCH, 1), lambda j: (0, 0)),
        out_shape=jax.ShapeDtypeStruct((BATCH, 1), jnp.float32),
        scratch_shapes=[
            pltpu.VMEM((BATCH, 1), jnp.float32),
        ],
    )(h, w2s, b2s, shift)


_NBT = BATCH // _BT


def _out_kernel(h_ref, w2_ref, b2_ref, lse_ref, out_ref):
    d = lax.dot_general(h_ref[...], w2_ref[...],
                        (((1,), (1,)), ((), ())),
                        preferred_element_type=jnp.float32)
    out_ref[...] = (d + b2_ref[...] - lse_ref[...]) * _LN2


def _compute_out(h, w2s, b2s, lse):
    return pl.pallas_call(
        _out_kernel,
        grid=(_VJ, _NBT),
        in_specs=[
            pl.BlockSpec((_BT, HIDDEN), lambda j, i: (i, 0)),
            pl.BlockSpec((_TVB, HIDDEN), lambda j, i: (j, 0)),
            pl.BlockSpec((1, _TVB), lambda j, i: (0, j)),
            pl.BlockSpec((_BT, 1), lambda j, i: (i, 0)),
        ],
        out_specs=pl.BlockSpec((_BT, _TVB), lambda j, i: (i, j)),
        out_shape=jax.ShapeDtypeStruct((BATCH, VOCAB), jnp.float32),
        compiler_params=pltpu.CompilerParams(
            dimension_semantics=("parallel", "parallel")),
    )(h, w2s, b2s, lse)


def kernel(inputs, emb, W1, b1, W2, b2):
    idx_flat = inputs.reshape(-1).astype(jnp.int32)
    emb_pad = jnp.pad(emb, ((0, 0), (0, _DPAD - EMBED_DIM)))
    embeds = _sc_gather(idx_flat, emb_pad)
    embeds = embeds.reshape(BATCH, CONTEXT * _DPAD)

    w1b = jnp.pad(
        W1.astype(jnp.bfloat16).reshape(HIDDEN, CONTEXT, EMBED_DIM),
        ((0, 0), (0, 0), (0, _DPAD - EMBED_DIM))).reshape(
            HIDDEN, CONTEXT * _DPAD)
    w2s = jnp.pad((W2 * _LOG2E).astype(jnp.bfloat16),
                  ((0, _VPAD - VOCAB), (0, 0)))
    b2s = jnp.pad(b2 * _LOG2E, (0, _VPAD - VOCAB),
                  constant_values=-1e30).reshape(1, _VPAD)

    h, shift = _compute_hidden(embeds, w1b, b1)
    lse = _compute_lse(h, w2s, b2s, shift)
    return _compute_out(h, w2s, b2s, lse)


# speedup vs baseline: 1.1757x; 1.1757x over previous
import functools

import jax
import jax.numpy as jnp
from jax import lax
from jax.experimental.layout import Format, Layout, with_layout_constraint
from jax.experimental import pallas as pl
from jax.experimental.pallas import tpu as pltpu
from jax.experimental.pallas import tpu_sc as plsc

VOCAB = 100000
EMBED_DIM = 64
CONTEXT = 20
BATCH = 4096
HIDDEN = 128

_NC = 2
_NS = 16
_NW = _NC * _NS

_CH = 128
_DPAD = 128

_VPAD = 100352
_TVA = 1024
_VJ = 16
_TVB = _VPAD // _VJ
_BT = 256

_LOG2E = 1.4426950408889634
_LN2 = 0.6931471805599453


def _sc_gather(idx_flat, emb_pad):
    n_idx = idx_flat.shape[0]
    d = emb_pad.shape[1]
    b_per_w = n_idx // _NW
    n_ch = b_per_w // _CH

    mesh = plsc.VectorSubcoreMesh(core_axis_name="c", subcore_axis_name="s")

    @functools.partial(
        pl.kernel,
        mesh=mesh,
        out_type=jax.ShapeDtypeStruct((n_idx, d), jnp.float32),
        scratch_types=[
            pltpu.VMEM((b_per_w,), jnp.int32),
            pltpu.VMEM((_CH, d), jnp.float32),
            pltpu.VMEM((_CH, d), jnp.float32),
            pltpu.SemaphoreType.DMA,
            pltpu.SemaphoreType.DMA,
        ],
    )
    def gather_kernel(table_hbm, idx_hbm, out_hbm, idx_v, buf0, buf1,
                      sem0, sem1):
        wid = lax.axis_index("s") * _NC + lax.axis_index("c")
        base = wid * b_per_w
        pltpu.sync_copy(idx_hbm.at[pl.ds(base, b_per_w)], idx_v)
        bufs = (buf0, buf1)
        sems = (sem0, sem1)
        copies = [None] * n_ch
        copies[0] = pltpu.async_copy(
            table_hbm.at[idx_v.at[pl.ds(0, _CH)]], bufs[0], sems[0])
        for t in range(n_ch):
            copies[t].wait()
            if t + 1 < n_ch:
                nb = (t + 1) % 2
                copies[t + 1] = pltpu.async_copy(
                    table_hbm.at[idx_v.at[pl.ds((t + 1) * _CH, _CH)]],
                    bufs[nb], sems[nb])
            pltpu.sync_copy(bufs[t % 2],
                            out_hbm.at[pl.ds(base + t * _CH, _CH)])

    return gather_kernel(emb_pad, idx_flat)


def _hidden_kernel(e_ref, w1_ref, b1_ref, h_ref, shift_ref):
    e = e_ref[...].astype(jnp.bfloat16)
    acc = lax.dot_general(e, w1_ref[...],
                          (((1,), (1,)), ((), ())),
                          preferred_element_type=jnp.float32)
    h = jnp.maximum(acc + b1_ref[...], 0.0)
    h_ref[...] = h.astype(jnp.bfloat16)
    bound = (jnp.sum(h, axis=1, keepdims=True) + 1.0) * (HIDDEN ** -0.5)
    shift_ref[...] = bound * _LOG2E


def _compute_hidden(embeds, w1b, b1):
    bt = 1024
    return pl.pallas_call(
        _hidden_kernel,
        grid=(BATCH // bt,),
        in_specs=[
            pl.BlockSpec((bt, CONTEXT * _DPAD), lambda i: (i, 0)),
            pl.BlockSpec((HIDDEN, CONTEXT * _DPAD), lambda i: (0, 0)),
            pl.BlockSpec((1, HIDDEN), lambda i: (0, 0)),
        ],
        out_specs=[
            pl.BlockSpec((bt, HIDDEN), lambda i: (i, 0)),
            pl.BlockSpec((bt, 1), lambda i: (i, 0)),
        ],
        out_shape=[
            jax.ShapeDtypeStruct((BATCH, HIDDEN), jnp.bfloat16),
            jax.ShapeDtypeStruct((BATCH, 1), jnp.float32),
        ],
    )(embeds, w1b, b1.reshape(1, HIDDEN))


def _lse_kernel(h_ref, w2_ref, b2_ref, shift_ref, lse_ref, s_s):
    j = pl.program_id(0)

    @pl.when(j == 0)
    def _init():
        s_s[...] = jnp.zeros((BATCH, 1), jnp.float32)

    d = lax.dot_general(h_ref[...], w2_ref[...],
                        (((1,), (1,)), ((), ())),
                        preferred_element_type=jnp.float32)
    e = jnp.exp2(d + b2_ref[...] - shift_ref[...])
    s_s[...] += jnp.sum(e, axis=1, keepdims=True)

    @pl.when(j == pl.num_programs(0) - 1)
    def _fin():
        lse_ref[...] = shift_ref[...] + jnp.log2(s_s[...])


def _compute_lse(h, w2s, b2s, shift):
    return pl.pallas_call(
        _lse_kernel,
        grid=(_VPAD // _TVA,),
        in_specs=[
            pl.BlockSpec((BATCH, HIDDEN), lambda j: (0, 0)),
            pl.BlockSpec((_TVA, HIDDEN), lambda j: (j, 0)),
            pl.BlockSpec((1, _TVA), lambda j: (0, j)),
            pl.BlockSpec((BATCH, 1), lambda j: (0, 0)),
        ],
        out_specs=pl.BlockSpec((BATCH, 1), lambda j: (0, 0)),
        out_shape=jax.ShapeDtypeStruct((BATCH, 1), jnp.float32),
        scratch_shapes=[
            pltpu.VMEM((BATCH, 1), jnp.float32),
        ],
    )(h, w2s, b2s, shift)


_NBT = BATCH // _BT


def _out_kernel(h_ref, w2_ref, b2_ref, lse_ref, out_ref):
    d = lax.dot_general(h_ref[...], w2_ref[...],
                        (((1,), (1,)), ((), ())),
                        preferred_element_type=jnp.float32)
    out_ref[...] = ((d + b2_ref[...] - lse_ref[...]) * _LN2).astype(
        jnp.bfloat16)


def _compute_out(h, w2s, b2s, lse):
    return pl.pallas_call(
        _out_kernel,
        grid=(_VJ, _NBT),
        in_specs=[
            pl.BlockSpec((_BT, HIDDEN), lambda j, i: (i, 0)),
            pl.BlockSpec((_TVB, HIDDEN), lambda j, i: (j, 0)),
            pl.BlockSpec((1, _TVB), lambda j, i: (0, j)),
            pl.BlockSpec((_BT, 1), lambda j, i: (i, 0)),
        ],
        out_specs=pl.BlockSpec((_BT, _TVB), lambda j, i: (i, j)),
        out_shape=jax.ShapeDtypeStruct((BATCH, VOCAB), jnp.bfloat16),
        compiler_params=pltpu.CompilerParams(
            dimension_semantics=("parallel", "parallel")),
    )(h, w2s, b2s, lse)


def kernel(inputs, emb, W1, b1, W2, b2):
    idx_flat = inputs.reshape(-1).astype(jnp.int32)
    emb_pad = jnp.pad(emb, ((0, 0), (0, _DPAD - EMBED_DIM)))
    embeds = _sc_gather(idx_flat, emb_pad)
    embeds = embeds.reshape(BATCH, CONTEXT * _DPAD)

    w1b = jnp.pad(
        W1.astype(jnp.bfloat16).reshape(HIDDEN, CONTEXT, EMBED_DIM),
        ((0, 0), (0, 0), (0, _DPAD - EMBED_DIM))).reshape(
            HIDDEN, CONTEXT * _DPAD)
    w2s = jnp.pad((W2 * _LOG2E).astype(jnp.bfloat16),
                  ((0, _VPAD - VOCAB), (0, 0)))
    b2s = jnp.pad(b2 * _LOG2E, (0, _VPAD - VOCAB),
                  constant_values=-1e30).reshape(1, _VPAD)

    h, shift = _compute_hidden(embeds, w1b, b1)
    lse = _compute_lse(h, w2s, b2s, shift)
    out = _compute_out(h, w2s, b2s, lse)
    return out.astype(jnp.float32)
